# baseline (speedup 1.0000x reference)
import jax
import jax.numpy as jnp
from jax import lax
from jax.experimental import pallas as pl
from jax.experimental.pallas import tpu as pltpu
from jax.experimental.pallas import tpu_sc as plsc

B = 16384
HIST = 200
D = 32
HID = 32
OUT = 5
NC, NS, L = 2, 16, 16
NW = NC * NS
BPW = B // NW
G1, G2 = 128, 72
UNROLL = 8
R = 16
C = 64
NCHUNK = BPW // C


def _sc_body(x_hbm, table_hbm, w1t_hbm, b1_hbm, w2t_hbm, b2_hbm, out_hbm,
             idx_v, row_v, w1t_v, b1_v, w2t_v, b2_v, out_v, gsems, isem):
    wid = lax.axis_index("s") * NC + lax.axis_index("c")
    base = wid * BPW

    pltpu.sync_copy(w1t_hbm, w1t_v)
    pltpu.sync_copy(b1_hbm, b1_v)
    pltpu.sync_copy(w2t_hbm, w2t_v)
    pltpu.sync_copy(b2_hbm, b2_v)

    def idx_chunk_copy(c):
        return pltpu.make_async_copy(
            x_hbm.at[pl.ds(base + c * C, C)], idx_v.at[c % 2], isem)

    idx_chunk_copy(0).start()
    idx_chunk_copy(0).wait()
    idx_chunk_copy(1).start()

    def gathers(b, slot):
        par = (b // C) % 2
        loc = b % C
        src = table_hbm.at[idx_v.at[par, loc, :]]
        sem = gsems.at[slot]
        return pltpu.make_async_copy(src, row_v.at[slot], sem)

    def issue(b, slot):
        gathers(b, slot).start()

    def drain(b, slot):
        gathers(b, slot).wait()

    zero = jnp.zeros((L,), jnp.float32)

    def process(b, slot):
        def i_body(i, accs):
            a0, a1, a2, a3 = accs
            r = i * UNROLL
            for u in range(UNROLL):
                row = row_v[slot, r + u, :]
                ev, od = plsc.unpack(row, format=plsc.PackFormat.INTERLEAVED)
                if u % 2 == 0:
                    a0 = a0 + ev
                    a1 = a1 + od
                else:
                    a2 = a2 + ev
                    a3 = a3 + od
            return (a0, a1, a2, a3)

        a0, a1, a2, a3 = lax.fori_loop(0, HIST // UNROLL, i_body,
                                       (zero, zero, zero, zero))
        s_ev, s_od = a0 + a2, a1 + a3

        e0 = b1_v[pl.ds(0, L)]
        f0 = b1_v[pl.ds(L, L)]
        e1 = zero
        f1 = zero
        for d in range(D):
            s = s_ev[d // 2] if d % 2 == 0 else s_od[d // 2]
            if d % 2 == 0:
                e0 = e0 + s * w1t_v[d, pl.ds(0, L)]
                f0 = f0 + s * w1t_v[d, pl.ds(L, L)]
            else:
                e1 = e1 + s * w1t_v[d, pl.ds(0, L)]
                f1 = f1 + s * w1t_v[d, pl.ds(L, L)]
        h_lo = jnp.maximum(e0 + e1, 0.0)
        h_hi = jnp.maximum(f0 + f1, 0.0)

        o0 = b2_v[pl.ds(0, L)]
        o1 = zero
        for j in range(HID):
            s = h_lo[j] if j < L else h_hi[j - L]
            if j % 2 == 0:
                o0 = o0 + s * w2t_v[j, pl.ds(0, L)]
            else:
                o1 = o1 + s * w2t_v[j, pl.ds(0, L)]
        out_v[b, pl.ds(0, L)] = o0 + o1

    for r in range(R):
        issue(r, r)

    def g_body(g, _):
        gb = g * R

        @pl.when(jnp.logical_and(g % (C // R) == 0, g > 0))
        def _():
            idx_chunk_copy(jnp.minimum(g // (C // R) + 1, NCHUNK - 1)).start()

        @pl.when(g % (C // R) == C // R - 1)
        def _():
            idx_chunk_copy(jnp.minimum(g // (C // R) + 1, NCHUNK - 1)).wait()

        for r in range(R):
            b = gb + r
            drain(b, r)
            process(b, r)
            issue(jnp.minimum(b + R, BPW - 1), r)
        return 0

    lax.fori_loop(0, BPW // R, g_body, 0)
    for r in range(R):
        drain(BPW - 1, r)

    pltpu.sync_copy(out_v, out_hbm.at[pl.ds(base, BPW)])


@jax.jit
def kernel(x, table, W1, b1, W2, b2):
    x32 = (x.astype(jnp.int32) % 4096)
    table_bf = table.astype(jnp.bfloat16)
    w1t = (W1.T * (1.0 / HIST)).astype(jnp.float32)
    w2t = jnp.zeros((HID, L), jnp.float32).at[:, :OUT].set(W2.T)
    b2p = jnp.zeros((L,), jnp.float32).at[:OUT].set(b2)

    mesh = plsc.VectorSubcoreMesh(core_axis_name="c", subcore_axis_name="s",
                                  num_cores=NC, num_subcores=NS)
    run = pl.kernel(
        _sc_body,
        out_type=jax.ShapeDtypeStruct((B, L), jnp.float32),
        mesh=mesh,
        compiler_params=pltpu.CompilerParams(use_tc_tiling_on_sc=False,
                                             needs_layout_passes=False),
        scratch_types=[
            pltpu.VMEM((2, C, HIST), jnp.int32),
            pltpu.VMEM((R, HIST, D), jnp.bfloat16),
            pltpu.VMEM((D, HID), jnp.float32),
            pltpu.VMEM((HID,), jnp.float32),
            pltpu.VMEM((HID, L), jnp.float32),
            pltpu.VMEM((L,), jnp.float32),
            pltpu.VMEM((BPW, L), jnp.float32),
            pltpu.SemaphoreType.DMA((R,)),
            pltpu.SemaphoreType.DMA,
        ],
    )
    padded = run(x32, table_bf, w1t, b1, w2t, b2p)
    return padded[:, :OUT]


# speedup vs baseline: 1.1802x; 1.1802x over previous
import jax
import jax.numpy as jnp
from jax import lax
from jax.experimental import pallas as pl
from jax.experimental.pallas import tpu as pltpu
from jax.experimental.pallas import tpu_sc as plsc

B = 16384
HIST = 200
D = 32
HID = 32
OUT = 5
NC, NS, L = 2, 16, 16
NW = NC * NS
BPW = B // NW
G1, G2 = 128, 72
UNROLL = 8
R = 8
C = 64
NCHUNK = BPW // C


def _sc_body(x_hbm, table_hbm, w1t_hbm, b1_hbm, w2t_hbm, b2_hbm, out_hbm,
             idx_v, row_v, w1t_v, b1_v, w2t_v, b2_v, out_v, gsems, isem):
    wid = lax.axis_index("s") * NC + lax.axis_index("c")
    base = wid * BPW

    pltpu.sync_copy(w1t_hbm, w1t_v)
    pltpu.sync_copy(b1_hbm, b1_v)
    pltpu.sync_copy(w2t_hbm, w2t_v)
    pltpu.sync_copy(b2_hbm, b2_v)

    def idx_chunk_copy(c):
        return pltpu.make_async_copy(
            x_hbm.at[pl.ds(base + c * C, C)], idx_v.at[c % 2], isem)

    idx_chunk_copy(0).start()
    idx_chunk_copy(0).wait()
    idx_chunk_copy(1).start()

    def gathers(b, slot):
        par = (b // C) % 2
        loc = b % C
        src = table_hbm.at[idx_v.at[par, loc, :]]
        sem = gsems.at[slot]
        return pltpu.make_async_copy(src, row_v.at[slot], sem)

    def issue(b, slot):
        gathers(b, slot).start()

    def drain(b, slot):
        gathers(b, slot).wait()

    zero = jnp.zeros((L,), jnp.float32)

    def process(b, slot):
        def i_body(i, accs):
            a0, a1, a2, a3 = accs
            r = i * UNROLL
            for u in range(UNROLL):
                lo = row_v[slot, r + u, pl.ds(0, L)]
                hi = row_v[slot, r + u, pl.ds(L, L)]
                if u % 2 == 0:
                    a0 = a0 + lo
                    a1 = a1 + hi
                else:
                    a2 = a2 + lo
                    a3 = a3 + hi
            return (a0, a1, a2, a3)

        a0, a1, a2, a3 = lax.fori_loop(0, HIST // UNROLL, i_body,
                                       (zero, zero, zero, zero))
        s_lo, s_hi = a0 + a2, a1 + a3

        e0 = b1_v[pl.ds(0, L)]
        f0 = b1_v[pl.ds(L, L)]
        e1 = zero
        f1 = zero
        for d in range(D):
            s = s_lo[d] if d < L else s_hi[d - L]
            if d % 2 == 0:
                e0 = e0 + s * w1t_v[d, pl.ds(0, L)]
                f0 = f0 + s * w1t_v[d, pl.ds(L, L)]
            else:
                e1 = e1 + s * w1t_v[d, pl.ds(0, L)]
                f1 = f1 + s * w1t_v[d, pl.ds(L, L)]
        h_lo = jnp.maximum(e0 + e1, 0.0)
        h_hi = jnp.maximum(f0 + f1, 0.0)

        o0 = b2_v[pl.ds(0, L)]
        o1 = zero
        for j in range(HID):
            s = h_lo[j] if j < L else h_hi[j - L]
            if j % 2 == 0:
                o0 = o0 + s * w2t_v[j, pl.ds(0, L)]
            else:
                o1 = o1 + s * w2t_v[j, pl.ds(0, L)]
        out_v[b, pl.ds(0, L)] = o0 + o1

    for r in range(R):
        issue(r, r)

    def g_body(g, _):
        gb = g * R

        @pl.when(jnp.logical_and(g % (C // R) == 0, g > 0))
        def _():
            idx_chunk_copy(jnp.minimum(g // (C // R) + 1, NCHUNK - 1)).start()

        @pl.when(g % (C // R) == C // R - 1)
        def _():
            idx_chunk_copy(jnp.minimum(g // (C // R) + 1, NCHUNK - 1)).wait()

        for r in range(R):
            b = gb + r
            drain(b, r)
            process(b, r)
            issue(jnp.minimum(b + R, BPW - 1), r)
        return 0

    lax.fori_loop(0, BPW // R, g_body, 0)
    for r in range(R):
        drain(BPW - 1, r)

    pltpu.sync_copy(out_v, out_hbm.at[pl.ds(base, BPW)])


@jax.jit
def kernel(x, table, W1, b1, W2, b2):
    x32 = x.astype(jnp.int32)
    w1t = (W1.T * (1.0 / HIST)).astype(jnp.float32)
    w2t = jnp.zeros((HID, L), jnp.float32).at[:, :OUT].set(W2.T)
    b2p = jnp.zeros((L,), jnp.float32).at[:OUT].set(b2)

    mesh = plsc.VectorSubcoreMesh(core_axis_name="c", subcore_axis_name="s",
                                  num_cores=NC, num_subcores=NS)
    run = pl.kernel(
        _sc_body,
        out_type=jax.ShapeDtypeStruct((B, L), jnp.float32),
        mesh=mesh,
        compiler_params=pltpu.CompilerParams(use_tc_tiling_on_sc=False,
                                             needs_layout_passes=False),
        scratch_types=[
            pltpu.VMEM((2, C, HIST), jnp.int32),
            pltpu.VMEM((R, HIST, D), jnp.float32),
            pltpu.VMEM((D, HID), jnp.float32),
            pltpu.VMEM((HID,), jnp.float32),
            pltpu.VMEM((HID, L), jnp.float32),
            pltpu.VMEM((L,), jnp.float32),
            pltpu.VMEM((BPW, L), jnp.float32),
            pltpu.SemaphoreType.DMA((R,)),
            pltpu.SemaphoreType.DMA,
        ],
    )
    padded = run(x32, table, w1t, b1, w2t, b2p)
    return padded[:, :OUT]


# speedup vs baseline: 1.1809x; 1.0006x over previous
import jax
import jax.numpy as jnp
from jax import lax
from jax.experimental import pallas as pl
from jax.experimental.pallas import tpu as pltpu
from jax.experimental.pallas import tpu_sc as plsc

B = 16384
HIST = 200
D = 32
HID = 32
OUT = 5
NC, NS, L = 2, 16, 16
NW = NC * NS
BPW = B // NW
UNROLL = 8
R = 8
C = 64
NCHUNK = BPW // C


def _sc_body(x_hbm, table_hbm, w1t_hbm, b1_hbm, w2t_hbm, b2_hbm, out_hbm,
             idx_v, row_v, w1t_v, b1_v, w2t_v, b2_v, out_v, gsems, isem):
    wid = lax.axis_index("s") * NC + lax.axis_index("c")
    base = wid * BPW

    pltpu.sync_copy(w1t_hbm, w1t_v)
    pltpu.sync_copy(b1_hbm, b1_v)
    pltpu.sync_copy(w2t_hbm, w2t_v)
    pltpu.sync_copy(b2_hbm, b2_v)

    def idx_chunk_copy(c):
        return pltpu.make_async_copy(
            x_hbm.at[pl.ds(base + c * C, C)], idx_v.at[c % 2], isem)

    idx_chunk_copy(0).start()
    idx_chunk_copy(0).wait()
    idx_chunk_copy(1).start()

    def gathers(b, slot):
        par = (b // C) % 2
        loc = b % C
        src = table_hbm.at[idx_v.at[par, loc, :]]
        sem = gsems.at[slot]
        return pltpu.make_async_copy(src, row_v.at[slot], sem)

    def issue(b, slot):
        gathers(b, slot).start()

    def drain(b, slot):
        gathers(b, slot).wait()

    zero = jnp.zeros((L,), jnp.float32)

    def process(b, slot):
        def i_body(i, accs):
            a0, a1, a2, a3 = accs
            r = i * UNROLL
            for u in range(UNROLL):
                lo = row_v[slot, r + u, pl.ds(0, L)]
                hi = row_v[slot, r + u, pl.ds(L, L)]
                if u % 2 == 0:
                    a0 = a0 + lo
                    a1 = a1 + hi
                else:
                    a2 = a2 + lo
                    a3 = a3 + hi
            return (a0, a1, a2, a3)

        a0, a1, a2, a3 = lax.fori_loop(0, HIST // UNROLL, i_body,
                                       (zero, zero, zero, zero))
        s_lo, s_hi = a0 + a2, a1 + a3

        e0 = b1_v[pl.ds(0, L)]
        f0 = b1_v[pl.ds(L, L)]
        e1 = zero
        f1 = zero
        for d in range(D):
            s = s_lo[d] if d < L else s_hi[d - L]
            if d % 2 == 0:
                e0 = e0 + s * w1t_v[d, pl.ds(0, L)]
                f0 = f0 + s * w1t_v[d, pl.ds(L, L)]
            else:
                e1 = e1 + s * w1t_v[d, pl.ds(0, L)]
                f1 = f1 + s * w1t_v[d, pl.ds(L, L)]
        h_lo = jnp.maximum(e0 + e1, 0.0)
        h_hi = jnp.maximum(f0 + f1, 0.0)

        o0 = b2_v[pl.ds(0, L)]
        o1 = zero
        for j in range(HID):
            s = h_lo[j] if j < L else h_hi[j - L]
            if j % 2 == 0:
                o0 = o0 + s * w2t_v[j, pl.ds(0, L)]
            else:
                o1 = o1 + s * w2t_v[j, pl.ds(0, L)]
        out_v[b, pl.ds(0, L)] = o0 + o1

    for r in range(R):
        issue(r, r)

    def g_body(g, _):
        gb = g * R

        @pl.when(jnp.logical_and(g % (C // R) == 0, g > 0))
        def _():
            idx_chunk_copy(jnp.minimum(g // (C // R) + 1, NCHUNK - 1)).start()

        @pl.when(g % (C // R) == C // R - 1)
        def _():
            idx_chunk_copy(jnp.minimum(g // (C // R) + 1, NCHUNK - 1)).wait()

        for r in range(R):
            b = gb + r
            drain(b, r)
            process(b, r)
            issue(jnp.minimum(b + R, BPW - 1), r)
        return 0

    lax.fori_loop(0, BPW // R, g_body, 0)
    for r in range(R):
        drain(BPW - 1, r)

    pltpu.sync_copy(out_v, out_hbm.at[pl.ds(base, BPW)])


@jax.jit
def kernel(x, table, W1, b1, W2, b2):
    x32 = x.astype(jnp.int32)
    w1t = (W1.T * (1.0 / HIST)).astype(jnp.float32)
    w2t = jnp.zeros((HID, L), jnp.float32).at[:, :OUT].set(W2.T)
    b2p = jnp.zeros((L,), jnp.float32).at[:OUT].set(b2)

    mesh = plsc.VectorSubcoreMesh(core_axis_name="c", subcore_axis_name="s",
                                  num_cores=NC, num_subcores=NS)
    run = pl.kernel(
        _sc_body,
        out_type=jax.ShapeDtypeStruct((B, L), jnp.float32),
        mesh=mesh,
        compiler_params=pltpu.CompilerParams(use_tc_tiling_on_sc=False,
                                             needs_layout_passes=False),
        scratch_types=[
            pltpu.VMEM((2, C, HIST), jnp.int32),
            pltpu.VMEM((R, HIST, D), jnp.float32),
            pltpu.VMEM((D, HID), jnp.float32),
            pltpu.VMEM((HID,), jnp.float32),
            pltpu.VMEM((HID, L), jnp.float32),
            pltpu.VMEM((L,), jnp.float32),
            pltpu.VMEM((BPW, L), jnp.float32),
            pltpu.SemaphoreType.DMA((R,)),
            pltpu.SemaphoreType.DMA,
        ],
    )
    padded = run(x32, table, w1t, b1, w2t, b2p)
    return padded[:, :OUT]
